# baseline (speedup 1.0000x reference)
import functools

import jax
import jax.numpy as jnp
from jax import lax
from jax.experimental import pallas as pl

_B, _L, _K = 256, 64, 6
_FIN, _FB, _FP = 64, 16, 256
_NB = 192
_RADIUS, _T, _OUT = 3, 2, 1


def _leaky(x):
    return jnp.where(x >= 0, x, 0.01 * x)


def _elu(x):
    return jnp.where(x > 0, x, jnp.exp(jnp.minimum(x, 0.0)) - 1.0)


def _mm(a, b):
    return jnp.dot(a, b, preferred_element_type=jnp.float32)


def _bmm(a, b):
    return lax.dot_general(a, b, (((2,), (1,)), ((0,), (0,))),
                           preferred_element_type=jnp.float32)


def _gru(x, h, wihT, whhT, bih, bhh):
    gi = _mm(x, wihT) + bih
    gh = _mm(h, whhT) + bhh
    r = jax.nn.sigmoid(gi[:, :_FP] + gh[:, :_FP])
    z = jax.nn.sigmoid(gi[:, _FP:2 * _FP] + gh[:, _FP:2 * _FP])
    n = jnp.tanh(gi[:, 2 * _FP:] + r * gh[:, 2 * _FP:])
    return (1.0 - z) * n + z * h


def _softmax_k(scores, att_masks):
    m = functools.reduce(jnp.maximum, scores)
    es = [jnp.exp(s - m) for s in scores]
    z = functools.reduce(jnp.add, es)
    return [e / z * am for e, am in zip(es, att_masks)]


def _body(atom_ref, bond_ref, adeg_ref, bdeg_ref, msub_ref,
          afc_wT_ref, afc_b_ref, nfa_wT_ref, nfb_wT_ref, nfc_b_ref,
          al_wa_ref, al_wn_ref, al_b_ref, att_wT_ref, att_b_ref,
          wih_ref, whh_ref, bih_ref, bhh_ref,
          mal_wm_ref, mal_wv_ref, mal_b_ref, matt_wT_ref, matt_b_ref,
          mwih_ref, mwhh_ref, mbih_ref, mbhh_ref,
          out_wT_ref, out_b_ref,
          atom_out_ref, pred_out_ref):
    bB = atom_ref.shape[0]
    R = bB * _L
    atoms = atom_ref[...].reshape(R, _FIN)
    adeg = adeg_ref[...]
    bdeg = bdeg_ref[...]

    iota_l = lax.broadcasted_iota(jnp.int32, (bB, _L, _L), 2)
    iota_nb = lax.broadcasted_iota(jnp.int32, (bB, _L, _NB), 2)
    onehot_a = []
    att_masks = []
    sm_masks = []
    for k in range(_K):
        idx_k = adeg[:, :, k:k + 1]
        onehot_a.append((idx_k == iota_l).astype(jnp.float32))
        att_masks.append((idx_k != _L - 1).astype(jnp.float32))
        sm_masks.append(jnp.where(idx_k == _L - 1, -9e8, 0.0))

    af = _leaky(_mm(atoms, afc_wT_ref[...]) + afc_b_ref[...])
    af3 = af.reshape(bB, _L, _FP)

    ap3 = _mm(atoms, nfa_wT_ref[...]).reshape(bB, _L, _FP)
    nfb_wT = nfb_wT_ref[...]
    nfc_b = nfc_b_ref[...]
    nf = []
    for k in range(_K):
        ga = _bmm(onehot_a[k], ap3)
        oh_b = (bdeg[:, :, k:k + 1] == iota_nb).astype(jnp.float32)
        gb = _bmm(oh_b, bond_ref[...])
        gbp = _mm(gb.reshape(R, _FB), nfb_wT).reshape(bB, _L, _FP)
        nf.append(_leaky(ga + gbp + nfc_b.reshape(1, 1, _FP)))

    wa3 = al_wa_ref[0:1, :].reshape(1, 1, _FP)
    wn3 = al_wn_ref[0:1, :].reshape(1, 1, _FP)
    adot = jnp.sum(af3 * wa3, axis=-1, keepdims=True)
    scores = [
        _leaky(adot + jnp.sum(nf[k] * wn3, axis=-1, keepdims=True)
               + al_b_ref[0, 0]) + sm_masks[k]
        for k in range(_K)
    ]
    attw = _softmax_k(scores, att_masks)
    sw = functools.reduce(jnp.add, attw)
    ns = functools.reduce(jnp.add, [w * f for w, f in zip(attw, nf)])
    ctx = _elu((_mm(ns.reshape(R, _FP), att_wT_ref[0])).reshape(bB, _L, _FP)
               + sw * att_b_ref[0:1, :].reshape(1, 1, _FP))
    h = _gru(ctx.reshape(R, _FP), af,
             wih_ref[0], whh_ref[0], bih_ref[0:1, :], bhh_ref[0:1, :])
    act = jnp.maximum(h, 0.0)

    for d in range(1, _RADIUS):
        wa3 = al_wa_ref[d:d + 1, :].reshape(1, 1, _FP)
        wn3 = al_wn_ref[d:d + 1, :].reshape(1, 1, _FP)
        act3 = act.reshape(bB, _L, _FP)
        adot = jnp.sum(act3 * wa3, axis=-1, keepdims=True)
        p3 = jnp.sum(act3 * wn3, axis=-1, keepdims=True)
        scores = [
            _leaky(adot + _bmm(onehot_a[k], p3) + al_b_ref[d, 0]) + sm_masks[k]
            for k in range(_K)
        ]
        attw = _softmax_k(scores, att_masks)
        sw = functools.reduce(jnp.add, attw)
        mix = functools.reduce(jnp.add,
                               [w * o for w, o in zip(attw, onehot_a)])
        ns = _bmm(mix, act3)
        ctx = _elu(_mm(ns.reshape(R, _FP), att_wT_ref[d]).reshape(bB, _L, _FP)
                   + sw * att_b_ref[d:d + 1, :].reshape(1, 1, _FP))
        h = _gru(ctx.reshape(R, _FP), h,
                 wih_ref[d], whh_ref[d], bih_ref[d:d + 1, :], bhh_ref[d:d + 1, :])
        act = jnp.maximum(h, 0.0)

    atom_out_ref[...] = h.reshape(bB, _L, _FP)

    msub = msub_ref[...]
    act3 = act.reshape(bB, _L, _FP)
    molf = jnp.sum(act3 * msub, axis=1)
    msm = jnp.where(msub == 0.0, -9e8, 0.0)
    wv3 = mal_wv_ref[...].reshape(1, 1, _FP)
    vdot = jnp.sum(act3 * wv3, axis=-1, keepdims=True)
    for _ in range(_T):
        amol = jnp.maximum(molf, 0.0)
        mdot = jnp.sum(amol * mal_wm_ref[...], axis=-1, keepdims=True)
        s = _leaky(mdot.reshape(bB, 1, 1) + vdot + mal_b_ref[0, 0]) + msm
        s = s - jnp.max(s, axis=1, keepdims=True)
        e = jnp.exp(s)
        mw = e / jnp.sum(e, axis=1, keepdims=True) * msub
        swm = jnp.sum(mw, axis=1)
        msum = jnp.sum(mw * act3, axis=1)
        mctx = _elu(_mm(msum, matt_wT_ref[...]) + swm * matt_b_ref[...])
        molf = _gru(mctx, molf, mwih_ref[...], mwhh_ref[...],
                    mbih_ref[...], mbhh_ref[...])
    pred_out_ref[...] = _mm(molf, out_wT_ref[...]) + out_b_ref[...]


def _run(atom_list, bond_list, adeg, bdeg, msub, weights, bB, interpret=False):
    grid = (_B // bB,)

    def blk(shape, imap):
        return pl.BlockSpec(shape, imap)

    rep3 = lambda i: (0, 0, 0)
    rep2 = lambda i: (0, 0)
    in_specs = [
        blk((bB, _L, _FIN), lambda i: (i, 0, 0)),
        blk((bB, _NB, _FB), lambda i: (i, 0, 0)),
        blk((bB, _L, _K), lambda i: (i, 0, 0)),
        blk((bB, _L, _K), lambda i: (i, 0, 0)),
        blk((bB, _L, 1), lambda i: (i, 0, 0)),
    ]
    for w in weights:
        in_specs.append(blk(w.shape, rep3 if w.ndim == 3 else rep2))

    out_shape = [
        jax.ShapeDtypeStruct((_B, _L, _FP), jnp.float32),
        jax.ShapeDtypeStruct((_B, _OUT), jnp.float32),
    ]
    out_specs = [
        blk((bB, _L, _FP), lambda i: (i, 0, 0)),
        blk((bB, _OUT), lambda i: (i, 0)),
    ]
    return pl.pallas_call(
        _body,
        grid=grid,
        in_specs=in_specs,
        out_specs=out_specs,
        out_shape=out_shape,
        interpret=interpret,
    )(atom_list, bond_list, adeg, bdeg, msub, *weights)


def _prep_and_run(atom_list, bond_list, atom_degree_list, bond_degree_list,
                  atom_mask, atom_fc_w, atom_fc_b, neighbor_fc_w, neighbor_fc_b,
                  align_w, align_b, attend_w, attend_b,
                  gru_wih, gru_whh, gru_bih, gru_bhh,
                  mol_align_w, mol_align_b, mol_attend_w, mol_attend_b,
                  mol_gru_wih, mol_gru_whh, mol_gru_bih, mol_gru_bhh,
                  out_w, out_b, interpret=False, bB=8):
    adeg = atom_degree_list.astype(jnp.int32)
    bdeg = bond_degree_list.astype(jnp.int32)
    msub = atom_mask.astype(jnp.float32).reshape(_B, _L, 1)
    weights = [
        atom_fc_w.T, atom_fc_b.reshape(1, _FP),
        neighbor_fc_w[:, :_FIN].T, neighbor_fc_w[:, _FIN:].T,
        neighbor_fc_b.reshape(1, _FP),
        align_w[:, 0, :_FP], align_w[:, 0, _FP:], align_b,
        jnp.transpose(attend_w, (0, 2, 1)), attend_b,
        jnp.transpose(gru_wih, (0, 2, 1)), jnp.transpose(gru_whh, (0, 2, 1)),
        gru_bih, gru_bhh,
        mol_align_w[:, :_FP], mol_align_w[:, _FP:], mol_align_b.reshape(1, 1),
        mol_attend_w.T, mol_attend_b.reshape(1, _FP),
        mol_gru_wih.T, mol_gru_whh.T,
        mol_gru_bih.reshape(1, 3 * _FP), mol_gru_bhh.reshape(1, 3 * _FP),
        out_w.T, out_b.reshape(1, _OUT),
    ]
    weights = [w.astype(jnp.float32) for w in weights]
    return _run(atom_list.astype(jnp.float32), bond_list.astype(jnp.float32),
                adeg, bdeg, msub, weights, bB, interpret=interpret)


@jax.jit
def kernel(atom_list, bond_list, atom_degree_list, bond_degree_list, atom_mask,
           atom_fc_w, atom_fc_b, neighbor_fc_w, neighbor_fc_b,
           align_w, align_b, attend_w, attend_b,
           gru_wih, gru_whh, gru_bih, gru_bhh,
           mol_align_w, mol_align_b, mol_attend_w, mol_attend_b,
           mol_gru_wih, mol_gru_whh, mol_gru_bih, mol_gru_bhh,
           out_w, out_b):
    atom_feature, mol_prediction = _prep_and_run(
        atom_list, bond_list, atom_degree_list, bond_degree_list, atom_mask,
        atom_fc_w, atom_fc_b, neighbor_fc_w, neighbor_fc_b,
        align_w, align_b, attend_w, attend_b,
        gru_wih, gru_whh, gru_bih, gru_bhh,
        mol_align_w, mol_align_b, mol_attend_w, mol_attend_b,
        mol_gru_wih, mol_gru_whh, mol_gru_bih, mol_gru_bhh, out_w, out_b)
    return atom_feature, mol_prediction


# speedup vs baseline: 2.2130x; 2.2130x over previous
import functools

import jax
import jax.numpy as jnp
from jax import lax
from jax.experimental import pallas as pl

_B, _L, _K = 256, 64, 6
_FIN, _FB, _FP = 64, 16, 256
_NB = 192
_RADIUS, _T, _OUT = 3, 2, 1


def _leaky(x):
    return jnp.where(x >= 0, x, 0.01 * x)


def _elu(x):
    return jnp.where(x > 0, x, jnp.exp(jnp.minimum(x, 0.0)) - 1.0)


def _mm(a, b):
    return jnp.dot(a, b, preferred_element_type=jnp.float32)


def _bmm(a, b):
    return lax.dot_general(a, b, (((2,), (1,)), ((0,), (0,))),
                           preferred_element_type=jnp.float32)


def _gru(x, h, wihT, whhT, bih, bhh):
    gi = _mm(x, wihT) + bih
    gh = _mm(h, whhT) + bhh
    r = jax.nn.sigmoid(gi[:, :_FP] + gh[:, :_FP])
    z = jax.nn.sigmoid(gi[:, _FP:2 * _FP] + gh[:, _FP:2 * _FP])
    n = jnp.tanh(gi[:, 2 * _FP:] + r * gh[:, 2 * _FP:])
    return (1.0 - z) * n + z * h


def _softmax_k(scores, att_masks):
    m = functools.reduce(jnp.maximum, scores)
    es = [jnp.exp(s - m) for s in scores]
    z = functools.reduce(jnp.add, es)
    return [e / z * am for e, am in zip(es, att_masks)]


def _body(atom_ref, bond_ref, adeg_ref, bdeg_ref, msub_ref,
          afc_wT_ref, afc_b_ref, nfa_wT_ref, nfb_wT_ref, nfc_b_ref,
          al_wa_ref, al_wn_ref, al_b_ref, att_wT_ref, att_b_ref,
          wih_ref, whh_ref, bih_ref, bhh_ref,
          mal_wm_ref, mal_wv_ref, mal_b_ref, matt_wT_ref, matt_b_ref,
          mwih_ref, mwhh_ref, mbih_ref, mbhh_ref,
          out_wT_ref, out_b_ref,
          atom_out_ref, pred_out_ref):
    bB = atom_ref.shape[0]
    R = bB * _L
    atoms = atom_ref[...].reshape(R, _FIN)
    adeg = adeg_ref[...]
    bdeg = bdeg_ref[...]

    iota_l = lax.broadcasted_iota(jnp.int32, (bB, _L, _L), 2)
    iota_nb = lax.broadcasted_iota(jnp.int32, (bB, _L, _NB), 2)
    onehot_a = []
    att_masks = []
    sm_masks = []
    for k in range(_K):
        idx_k = adeg[:, :, k:k + 1]
        onehot_a.append((idx_k == iota_l).astype(jnp.float32))
        att_masks.append((idx_k != _L - 1).astype(jnp.float32))
        sm_masks.append(jnp.where(idx_k == _L - 1, -9e8, 0.0))

    af = _leaky(_mm(atoms, afc_wT_ref[...]) + afc_b_ref[...])
    af3 = af.reshape(bB, _L, _FP)

    ap3 = _mm(atoms, nfa_wT_ref[...]).reshape(bB, _L, _FP)
    nfb_wT = nfb_wT_ref[...]
    nfc_b = nfc_b_ref[...]
    nf = []
    for k in range(_K):
        ga = _bmm(onehot_a[k], ap3)
        oh_b = (bdeg[:, :, k:k + 1] == iota_nb).astype(jnp.float32)
        gb = _bmm(oh_b, bond_ref[...])
        gbp = _mm(gb.reshape(R, _FB), nfb_wT).reshape(bB, _L, _FP)
        nf.append(_leaky(ga + gbp + nfc_b.reshape(1, 1, _FP)))

    wa3 = al_wa_ref[0:1, :].reshape(1, 1, _FP)
    wn3 = al_wn_ref[0:1, :].reshape(1, 1, _FP)
    adot = jnp.sum(af3 * wa3, axis=-1, keepdims=True)
    scores = [
        _leaky(adot + jnp.sum(nf[k] * wn3, axis=-1, keepdims=True)
               + al_b_ref[0, 0]) + sm_masks[k]
        for k in range(_K)
    ]
    attw = _softmax_k(scores, att_masks)
    sw = functools.reduce(jnp.add, attw)
    ns = functools.reduce(jnp.add, [w * f for w, f in zip(attw, nf)])
    ctx = _elu((_mm(ns.reshape(R, _FP), att_wT_ref[0])).reshape(bB, _L, _FP)
               + sw * att_b_ref[0:1, :].reshape(1, 1, _FP))
    h = _gru(ctx.reshape(R, _FP), af,
             wih_ref[0], whh_ref[0], bih_ref[0:1, :], bhh_ref[0:1, :])
    act = jnp.maximum(h, 0.0)

    for d in range(1, _RADIUS):
        wa3 = al_wa_ref[d:d + 1, :].reshape(1, 1, _FP)
        wn3 = al_wn_ref[d:d + 1, :].reshape(1, 1, _FP)
        act3 = act.reshape(bB, _L, _FP)
        adot = jnp.sum(act3 * wa3, axis=-1, keepdims=True)
        p3 = jnp.sum(act3 * wn3, axis=-1, keepdims=True)
        scores = [
            _leaky(adot + _bmm(onehot_a[k], p3) + al_b_ref[d, 0]) + sm_masks[k]
            for k in range(_K)
        ]
        attw = _softmax_k(scores, att_masks)
        sw = functools.reduce(jnp.add, attw)
        mix = functools.reduce(jnp.add,
                               [w * o for w, o in zip(attw, onehot_a)])
        ns = _bmm(mix, act3)
        ctx = _elu(_mm(ns.reshape(R, _FP), att_wT_ref[d]).reshape(bB, _L, _FP)
                   + sw * att_b_ref[d:d + 1, :].reshape(1, 1, _FP))
        h = _gru(ctx.reshape(R, _FP), h,
                 wih_ref[d], whh_ref[d], bih_ref[d:d + 1, :], bhh_ref[d:d + 1, :])
        act = jnp.maximum(h, 0.0)

    atom_out_ref[...] = h.reshape(bB, _L, _FP)

    msub = msub_ref[...]
    act3 = act.reshape(bB, _L, _FP)
    molf = jnp.sum(act3 * msub, axis=1)
    msm = jnp.where(msub == 0.0, -9e8, 0.0)
    wv3 = mal_wv_ref[...].reshape(1, 1, _FP)
    vdot = jnp.sum(act3 * wv3, axis=-1, keepdims=True)
    for _ in range(_T):
        amol = jnp.maximum(molf, 0.0)
        mdot = jnp.sum(amol * mal_wm_ref[...], axis=-1, keepdims=True)
        s = _leaky(mdot.reshape(bB, 1, 1) + vdot + mal_b_ref[0, 0]) + msm
        s = s - jnp.max(s, axis=1, keepdims=True)
        e = jnp.exp(s)
        mw = e / jnp.sum(e, axis=1, keepdims=True) * msub
        swm = jnp.sum(mw, axis=1)
        msum = jnp.sum(mw * act3, axis=1)
        mctx = _elu(_mm(msum, matt_wT_ref[...]) + swm * matt_b_ref[...])
        molf = _gru(mctx, molf, mwih_ref[...], mwhh_ref[...],
                    mbih_ref[...], mbhh_ref[...])
    pred_out_ref[...] = _mm(molf, out_wT_ref[...]) + out_b_ref[...]


def _run(atom_list, bond_list, adeg, bdeg, msub, weights, bB, interpret=False):
    grid = (_B // bB,)

    def blk(shape, imap):
        return pl.BlockSpec(shape, imap)

    rep3 = lambda i: (0, 0, 0)
    rep2 = lambda i: (0, 0)
    in_specs = [
        blk((bB, _L, _FIN), lambda i: (i, 0, 0)),
        blk((bB, _NB, _FB), lambda i: (i, 0, 0)),
        blk((bB, _L, _K), lambda i: (i, 0, 0)),
        blk((bB, _L, _K), lambda i: (i, 0, 0)),
        blk((bB, _L, 1), lambda i: (i, 0, 0)),
    ]
    for w in weights:
        in_specs.append(blk(w.shape, rep3 if w.ndim == 3 else rep2))

    out_shape = [
        jax.ShapeDtypeStruct((_B, _L, _FP), jnp.float32),
        jax.ShapeDtypeStruct((_B, _OUT), jnp.float32),
    ]
    out_specs = [
        blk((bB, _L, _FP), lambda i: (i, 0, 0)),
        blk((bB, _OUT), lambda i: (i, 0)),
    ]
    return pl.pallas_call(
        _body,
        grid=grid,
        in_specs=in_specs,
        out_specs=out_specs,
        out_shape=out_shape,
        interpret=interpret,
    )(atom_list, bond_list, adeg, bdeg, msub, *weights)


def _prep_and_run(atom_list, bond_list, atom_degree_list, bond_degree_list,
                  atom_mask, atom_fc_w, atom_fc_b, neighbor_fc_w, neighbor_fc_b,
                  align_w, align_b, attend_w, attend_b,
                  gru_wih, gru_whh, gru_bih, gru_bhh,
                  mol_align_w, mol_align_b, mol_attend_w, mol_attend_b,
                  mol_gru_wih, mol_gru_whh, mol_gru_bih, mol_gru_bhh,
                  out_w, out_b, interpret=False, bB=16):
    adeg = atom_degree_list.astype(jnp.int32)
    bdeg = bond_degree_list.astype(jnp.int32)
    msub = atom_mask.astype(jnp.float32).reshape(_B, _L, 1)
    weights = [
        atom_fc_w.T, atom_fc_b.reshape(1, _FP),
        neighbor_fc_w[:, :_FIN].T, neighbor_fc_w[:, _FIN:].T,
        neighbor_fc_b.reshape(1, _FP),
        align_w[:, 0, :_FP], align_w[:, 0, _FP:], align_b,
        jnp.transpose(attend_w, (0, 2, 1)), attend_b,
        jnp.transpose(gru_wih, (0, 2, 1)), jnp.transpose(gru_whh, (0, 2, 1)),
        gru_bih, gru_bhh,
        mol_align_w[:, :_FP], mol_align_w[:, _FP:], mol_align_b.reshape(1, 1),
        mol_attend_w.T, mol_attend_b.reshape(1, _FP),
        mol_gru_wih.T, mol_gru_whh.T,
        mol_gru_bih.reshape(1, 3 * _FP), mol_gru_bhh.reshape(1, 3 * _FP),
        out_w.T, out_b.reshape(1, _OUT),
    ]
    weights = [w.astype(jnp.float32) for w in weights]
    return _run(atom_list.astype(jnp.float32), bond_list.astype(jnp.float32),
                adeg, bdeg, msub, weights, bB, interpret=interpret)


@jax.jit
def kernel(atom_list, bond_list, atom_degree_list, bond_degree_list, atom_mask,
           atom_fc_w, atom_fc_b, neighbor_fc_w, neighbor_fc_b,
           align_w, align_b, attend_w, attend_b,
           gru_wih, gru_whh, gru_bih, gru_bhh,
           mol_align_w, mol_align_b, mol_attend_w, mol_attend_b,
           mol_gru_wih, mol_gru_whh, mol_gru_bih, mol_gru_bhh,
           out_w, out_b):
    atom_feature, mol_prediction = _prep_and_run(
        atom_list, bond_list, atom_degree_list, bond_degree_list, atom_mask,
        atom_fc_w, atom_fc_b, neighbor_fc_w, neighbor_fc_b,
        align_w, align_b, attend_w, attend_b,
        gru_wih, gru_whh, gru_bih, gru_bhh,
        mol_align_w, mol_align_b, mol_attend_w, mol_attend_b,
        mol_gru_wih, mol_gru_whh, mol_gru_bih, mol_gru_bhh, out_w, out_b)
    return atom_feature, mol_prediction
